# baseline (speedup 1.0000x reference)
import functools

import jax
import jax.numpy as jnp
from jax import lax
from jax.experimental import pallas as pl
from jax.experimental.pallas import tpu as pltpu
from jax.experimental.pallas import tpu_sc as plsc

N = 50000
D = 64
NREL = 6
RBLK = 512
NPAD = 50176
GRID = NPAD // RBLK
EPS = 1e-5
CH = 128
SUB = 16
HALF = 32


def _gn(x, g, b):
    mu = jnp.mean(x, axis=1, keepdims=True)
    xc = x - mu
    var = jnp.mean(xc * xc, axis=1, keepdims=True)
    return g * xc * jax.lax.rsqrt(var + EPS) + b


def _full_spec(shape):
    return pl.BlockSpec(shape, lambda i: tuple(0 for _ in shape))


def _k1_body(ctrs_ref, feats_ref, win1, bin1, win2, gin, bein,
             wseg1, bseg1, wseg2, gseg, beseg, wall,
             feat_ref, t0_ref, ya_ref, yb_ref):
    ctrs = ctrs_ref[...]
    feats = feats_ref[...]
    w1 = win1[...]
    x1 = jnp.maximum(ctrs[:, 0:1] * w1[0:1, :] + ctrs[:, 1:2] * w1[1:2, :]
                     + bin1[...], 0.0)
    h1 = _gn(jnp.dot(x1, win2[...], preferred_element_type=jnp.float32),
             gin[...], bein[...])
    w2 = wseg1[...]
    x2 = jnp.maximum(feats[:, 0:1] * w2[0:1, :] + feats[:, 1:2] * w2[1:2, :]
                     + bseg1[...], 0.0)
    h2 = _gn(jnp.dot(x2, wseg2[...], preferred_element_type=jnp.float32),
             gseg[...], beseg[...])
    f = jnp.maximum(h1 + h2, 0.0)
    feat_ref[...] = f
    y = jnp.dot(f, wall[...], preferred_element_type=jnp.float32)
    t0_ref[...] = y[:, 0:D]
    for r in range(NREL):
        base = D + D * r
        ya_ref[r] = _pack(y[:, base:base + HALF])
        yb_ref[r] = _pack(y[:, base + HALF:base + D])


def _pack(ys):
    q = RBLK // 4
    return jnp.concatenate([ys[0:q], ys[q:2 * q], ys[2 * q:3 * q],
                            ys[3 * q:4 * q]], axis=1)


def _make_k1():
    row = lambda w: pl.BlockSpec((RBLK, w), lambda i: (i, 0))
    in_specs = [
        row(2), row(2),
        _full_spec((2, D)), _full_spec((1, D)), _full_spec((D, D)),
        _full_spec((1, D)), _full_spec((1, D)),
        _full_spec((2, D)), _full_spec((1, D)), _full_spec((D, D)),
        _full_spec((1, D)), _full_spec((1, D)),
        _full_spec((D, D * (NREL + 1))),
    ]
    out_specs = [
        row(D), row(D),
        pl.BlockSpec((NREL, RBLK // 4, 4 * HALF), lambda i: (0, i, 0)),
        pl.BlockSpec((NREL, RBLK // 4, 4 * HALF), lambda i: (0, i, 0)),
    ]
    out_shape = [
        jax.ShapeDtypeStruct((NPAD, D), jnp.float32),
        jax.ShapeDtypeStruct((NPAD, D), jnp.float32),
        jax.ShapeDtypeStruct((NREL, NPAD // 4, 4 * HALF), jnp.float32),
        jax.ShapeDtypeStruct((NREL, NPAD // 4, 4 * HALF), jnp.float32),
    ]
    return pl.pallas_call(_k1_body, grid=(GRID,), in_specs=in_specs,
                          out_specs=out_specs, out_shape=out_shape)


def _blk_common(sa_ref, sb_ref, t0_ref, res_ref, gn_g, gn_b, wc2, g2, b2):
    temp = t0_ref[...] + jnp.concatenate([sa_ref[...], sb_ref[...]], axis=1)
    x = jnp.maximum(_gn(temp, gn_g[...], gn_b[...]), 0.0)
    y2 = jnp.dot(x, wc2[...], preferred_element_type=jnp.float32)
    z = _gn(y2, g2[...], b2[...]) + res_ref[...]
    return jnp.maximum(z, 0.0)


def _blk_mid_body(sa_ref, sb_ref, t0_ref, res_ref, gn_g, gn_b, wc2, g2, b2,
                  wall, feat_ref, t0_o, ya_o, yb_o):
    f = _blk_common(sa_ref, sb_ref, t0_ref, res_ref, gn_g, gn_b, wc2, g2, b2)
    feat_ref[...] = f
    y = jnp.dot(f, wall[...], preferred_element_type=jnp.float32)
    t0_o[...] = y[:, 0:D]
    for r in range(NREL):
        base = D + D * r
        ya_o[r] = _pack(y[:, base:base + HALF])
        yb_o[r] = _pack(y[:, base + HALF:base + D])


def _blk_last_body(sa_ref, sb_ref, t0_ref, res_ref, gn_g, gn_b, wc2, g2, b2,
                   out_ref):
    out_ref[...] = _blk_common(sa_ref, sb_ref, t0_ref, res_ref, gn_g, gn_b,
                               wc2, g2, b2)


def _make_blk(last):
    row = lambda w: pl.BlockSpec((RBLK, w), lambda i: (i, 0))
    in_specs = [
        row(HALF), row(HALF), row(D), row(D),
        _full_spec((1, D)), _full_spec((1, D)), _full_spec((D, D)),
        _full_spec((1, D)), _full_spec((1, D)),
    ]
    if last:
        out_specs = row(D)
        out_shape = jax.ShapeDtypeStruct((N, D), jnp.float32)
        return pl.pallas_call(_blk_last_body, grid=(GRID,), in_specs=in_specs,
                              out_specs=out_specs, out_shape=out_shape)
    in_specs.append(_full_spec((D, D * (NREL + 1))))
    out_specs = [
        row(D), row(D),
        pl.BlockSpec((NREL, RBLK // 4, 4 * HALF), lambda i: (0, i, 0)),
        pl.BlockSpec((NREL, RBLK // 4, 4 * HALF), lambda i: (0, i, 0)),
    ]
    out_shape = [
        jax.ShapeDtypeStruct((NPAD, D), jnp.float32),
        jax.ShapeDtypeStruct((NPAD, D), jnp.float32),
        jax.ShapeDtypeStruct((NREL, NPAD // 4, 4 * HALF), jnp.float32),
        jax.ShapeDtypeStruct((NREL, NPAD // 4, 4 * HALF), jnp.float32),
    ]
    return pl.pallas_call(_blk_mid_body, grid=(GRID,), in_specs=in_specs,
                          out_specs=out_specs, out_shape=out_shape)


CPK = 48
NKC = 17


def _make_sc_agg(epad):
    nch = epad // (SUB * CH)
    assert nch == CPK * NKC
    rpt = NPAD // SUB
    mesh = plsc.VectorSubcoreMesh(core_axis_name="c", subcore_axis_name="s")

    @functools.partial(
        pl.kernel, mesh=mesh,
        compiler_params=pltpu.CompilerParams(use_tc_tiling_on_sc=False),
        out_type=(jax.ShapeDtypeStruct((NPAD, HALF), jnp.float32),
                  jax.ShapeDtypeStruct((NPAD, HALF), jnp.float32)),
        scratch_types=[
            pltpu.VMEM((CPK, CH), jnp.int32),
            pltpu.VMEM((CPK, CH), jnp.int32),
            pltpu.VMEM((3, CH, HALF), jnp.float32),
            pltpu.VMEM_SHARED((NPAD, HALF), jnp.float32),
        ] + [pltpu.SemaphoreType.DMA] * 6,
    )
    def sc_agg(u_hbm, v_hbm, ya_hbm, yb_hbm, zero_hbm, outa_hbm, outb_hbm,
               u_k, v_k, rows_v, acc, *sems):
        c = lax.axis_index("c")
        s = lax.axis_index("s")
        rbase = s * rpt
        crow_base = s * nch

        def run(y_t, out_t):
            pltpu.sync_copy(zero_hbm.at[pl.ds(rbase, rpt)],
                            acc.at[pl.ds(rbase, rpt)])
            plsc.subcore_barrier()

            semg = sems[:3]
            semsc = sems[3:]
            for kc in range(NKC):
                pltpu.sync_copy(u_hbm.at[pl.ds(crow_base + kc * CPK, CPK)], u_k)
                pltpu.sync_copy(v_hbm.at[pl.ds(crow_base + kc * CPK, CPK)], v_k)
                pltpu.async_copy(y_t.at[v_k.at[0]], rows_v.at[0], semg[0])

                def inner(t, carry):
                    for b in (0, 1, 2):
                        j = t * 3 + b
                        bn = (b + 1) % 3
                        pltpu.make_async_copy(y_t.at[pl.ds(0, CH)],
                                              rows_v.at[b], semg[b]).wait()

                        @pl.when(j >= 2)
                        def _():
                            pltpu.make_async_copy(rows_v.at[bn],
                                                  acc.at[pl.ds(0, CH)],
                                                  semsc[bn]).wait()

                        @pl.when(j + 1 < CPK)
                        def _():
                            pltpu.async_copy(y_t.at[v_k.at[j + 1]],
                                             rows_v.at[bn], semg[bn])

                        pltpu.async_copy(rows_v.at[b], acc.at[u_k.at[j]],
                                         semsc[b], add=True)
                    return carry

                lax.fori_loop(0, CPK // 3, inner, 0)
                pltpu.make_async_copy(rows_v.at[(CPK - 2) % 3],
                                      acc.at[pl.ds(0, CH)],
                                      semsc[(CPK - 2) % 3]).wait()
                pltpu.make_async_copy(rows_v.at[(CPK - 1) % 3],
                                      acc.at[pl.ds(0, CH)],
                                      semsc[(CPK - 1) % 3]).wait()

            plsc.subcore_barrier()
            pltpu.sync_copy(acc.at[pl.ds(rbase, rpt)],
                            out_t.at[pl.ds(rbase, rpt)])

        @pl.when(c == 0)
        def _():
            run(ya_hbm, outa_hbm)

        @pl.when(c == 1)
        def _():
            run(yb_hbm, outb_hbm)

    return sc_agg


def kernel(feats, ctrs, W_in1, b_in1, W_in2, g_in, be_in,
           W_seg1, b_seg1, W_seg2, g_seg, be_seg,
           W_ctr, W_pre0, W_pre1, W_suc0, W_suc1, W_left, W_right, W_ctr2,
           g_norm, be_norm, g_ctr2, be_ctr2,
           pre0_u, pre0_v, suc0_u, suc0_v, pre1_u, pre1_v, suc1_u, suc1_v,
           left_u, left_v, right_u, right_v):
    f32 = jnp.float32
    r2 = lambda a: a.reshape(1, D).astype(f32)

    us = [pre0_u, suc0_u, pre1_u, suc1_u, left_u, right_u]
    vs = [pre0_v, suc0_v, pre1_v, suc1_v, left_v, right_v]
    e_tot = sum(int(u.shape[0]) for u in us)
    align = SUB * CH * CPK
    epad = ((e_tot + align - 1) // align) * align
    pad = epad - e_tot
    u_all = jnp.concatenate(
        [u.astype(jnp.int32) for u in us]
        + [jnp.full((pad,), N, jnp.int32)]).reshape(epad // CH, CH)
    def enc(v, r):
        v = v.astype(jnp.int32)
        l = v % jnp.int32(RBLK)
        return (jnp.int32(r * NPAD) + (v - l) + 4 * (l % jnp.int32(RBLK // 4))
                + l // jnp.int32(RBLK // 4))

    v_all = jnp.concatenate(
        [enc(v, r) for r, v in enumerate(vs)]
        + [jnp.zeros((pad,), jnp.int32)]).reshape(epad // CH, CH)
    zero = jnp.zeros((NPAD, HALF), jnp.float32)

    ctrs_p = jnp.zeros((NPAD, 2), f32).at[:N].set(ctrs)
    feats_p = jnp.zeros((NPAD, 2), f32).at[:N].set(feats)

    def wall(i):
        return jnp.concatenate(
            [W_ctr[i], W_pre0[i], W_suc0[i], W_pre1[i], W_suc1[i],
             W_left[i], W_right[i]], axis=1)

    k1 = _make_k1()
    blk_mid = _make_blk(last=False)
    blk_last = _make_blk(last=True)
    sc_agg = _make_sc_agg(epad)

    feat0, t00, ya, yb = k1(
        ctrs_p, feats_p, W_in1, r2(b_in1), W_in2, r2(g_in), r2(be_in),
        W_seg1, r2(b_seg1), W_seg2, r2(g_seg), r2(be_seg), wall(0))

    s0a, s0b = sc_agg(u_all, v_all,
                      ya.reshape(NREL * NPAD, HALF),
                      yb.reshape(NREL * NPAD, HALF), zero)

    feat1, t01, y1a, y1b = blk_mid(
        s0a, s0b, t00, feat0, r2(g_norm[0]), r2(be_norm[0]), W_ctr2[0],
        r2(g_ctr2[0]), r2(be_ctr2[0]), wall(1))

    s1a, s1b = sc_agg(u_all, v_all,
                      y1a.reshape(NREL * NPAD, HALF),
                      y1b.reshape(NREL * NPAD, HALF), zero)

    out = blk_last(s1a, s1b, t01, feat1, r2(g_norm[1]), r2(be_norm[1]),
                   W_ctr2[1], r2(g_ctr2[1]), r2(be_ctr2[1]))
    return out


# speedup vs baseline: 1.6141x; 1.6141x over previous
import functools

import jax
import jax.numpy as jnp
from jax import lax
from jax.experimental import pallas as pl
from jax.experimental.pallas import tpu as pltpu
from jax.experimental.pallas import tpu_sc as plsc

N = 50000
D = 64
NREL = 6
RBLK = 512
NPAD = 50176
GRID = NPAD // RBLK
EPS = 1e-5
CH = 128
SUB = 16
HALF = 32


def _gn(x, g, b):
    mu = jnp.mean(x, axis=1, keepdims=True)
    xc = x - mu
    var = jnp.mean(xc * xc, axis=1, keepdims=True)
    return g * xc * jax.lax.rsqrt(var + EPS) + b


def _full_spec(shape):
    return pl.BlockSpec(shape, lambda i: tuple(0 for _ in shape))


def _k1_body(ctrs_ref, feats_ref, win1, bin1, win2, gin, bein,
             wseg1, bseg1, wseg2, gseg, beseg, wall,
             feat_ref, t0_ref, ya_ref, yb_ref):
    ctrs = ctrs_ref[...]
    feats = feats_ref[...]
    w1 = win1[...]
    x1 = jnp.maximum(ctrs[:, 0:1] * w1[0:1, :] + ctrs[:, 1:2] * w1[1:2, :]
                     + bin1[...], 0.0)
    h1 = _gn(jnp.dot(x1, win2[...], preferred_element_type=jnp.float32),
             gin[...], bein[...])
    w2 = wseg1[...]
    x2 = jnp.maximum(feats[:, 0:1] * w2[0:1, :] + feats[:, 1:2] * w2[1:2, :]
                     + bseg1[...], 0.0)
    h2 = _gn(jnp.dot(x2, wseg2[...], preferred_element_type=jnp.float32),
             gseg[...], beseg[...])
    f = jnp.maximum(h1 + h2, 0.0)
    feat_ref[...] = f
    y = jnp.dot(f, wall[...], preferred_element_type=jnp.float32)
    t0_ref[...] = y[:, 0:D]
    for r in range(NREL):
        base = D + D * r
        ya_ref[r] = _pack(y[:, base:base + HALF])
        yb_ref[r] = _pack(y[:, base + HALF:base + D])


def _pack(ys):
    q = RBLK // 4
    return jnp.concatenate([ys[0:q], ys[q:2 * q], ys[2 * q:3 * q],
                            ys[3 * q:4 * q]], axis=1)


def _make_k1():
    row = lambda w: pl.BlockSpec((RBLK, w), lambda i: (i, 0))
    in_specs = [
        row(2), row(2),
        _full_spec((2, D)), _full_spec((1, D)), _full_spec((D, D)),
        _full_spec((1, D)), _full_spec((1, D)),
        _full_spec((2, D)), _full_spec((1, D)), _full_spec((D, D)),
        _full_spec((1, D)), _full_spec((1, D)),
        _full_spec((D, D * (NREL + 1))),
    ]
    out_specs = [
        row(D), row(D),
        pl.BlockSpec((NREL, RBLK // 4, 4 * HALF), lambda i: (0, i, 0)),
        pl.BlockSpec((NREL, RBLK // 4, 4 * HALF), lambda i: (0, i, 0)),
    ]
    out_shape = [
        jax.ShapeDtypeStruct((NPAD, D), jnp.float32),
        jax.ShapeDtypeStruct((NPAD, D), jnp.float32),
        jax.ShapeDtypeStruct((NREL, NPAD // 4, 4 * HALF), jnp.float32),
        jax.ShapeDtypeStruct((NREL, NPAD // 4, 4 * HALF), jnp.float32),
    ]
    return pl.pallas_call(_k1_body, grid=(GRID,), in_specs=in_specs,
                          out_specs=out_specs, out_shape=out_shape)


def _blk_common(sa_ref, sb_ref, t0_ref, res_ref, gn_g, gn_b, wc2, g2, b2):
    temp = t0_ref[...] + jnp.concatenate([sa_ref[...], sb_ref[...]], axis=1)
    x = jnp.maximum(_gn(temp, gn_g[...], gn_b[...]), 0.0)
    y2 = jnp.dot(x, wc2[...], preferred_element_type=jnp.float32)
    z = _gn(y2, g2[...], b2[...]) + res_ref[...]
    return jnp.maximum(z, 0.0)


def _blk_mid_body(sa_ref, sb_ref, t0_ref, res_ref, gn_g, gn_b, wc2, g2, b2,
                  wall, feat_ref, t0_o, ya_o, yb_o):
    f = _blk_common(sa_ref, sb_ref, t0_ref, res_ref, gn_g, gn_b, wc2, g2, b2)
    feat_ref[...] = f
    y = jnp.dot(f, wall[...], preferred_element_type=jnp.float32)
    t0_o[...] = y[:, 0:D]
    for r in range(NREL):
        base = D + D * r
        ya_o[r] = _pack(y[:, base:base + HALF])
        yb_o[r] = _pack(y[:, base + HALF:base + D])


def _blk_last_body(sa_ref, sb_ref, t0_ref, res_ref, gn_g, gn_b, wc2, g2, b2,
                   out_ref):
    out_ref[...] = _blk_common(sa_ref, sb_ref, t0_ref, res_ref, gn_g, gn_b,
                               wc2, g2, b2)


def _make_blk(last):
    row = lambda w: pl.BlockSpec((RBLK, w), lambda i: (i, 0))
    in_specs = [
        row(HALF), row(HALF), row(D), row(D),
        _full_spec((1, D)), _full_spec((1, D)), _full_spec((D, D)),
        _full_spec((1, D)), _full_spec((1, D)),
    ]
    if last:
        out_specs = row(D)
        out_shape = jax.ShapeDtypeStruct((N, D), jnp.float32)
        return pl.pallas_call(_blk_last_body, grid=(GRID,), in_specs=in_specs,
                              out_specs=out_specs, out_shape=out_shape)
    in_specs.append(_full_spec((D, D * (NREL + 1))))
    out_specs = [
        row(D), row(D),
        pl.BlockSpec((NREL, RBLK // 4, 4 * HALF), lambda i: (0, i, 0)),
        pl.BlockSpec((NREL, RBLK // 4, 4 * HALF), lambda i: (0, i, 0)),
    ]
    out_shape = [
        jax.ShapeDtypeStruct((NPAD, D), jnp.float32),
        jax.ShapeDtypeStruct((NPAD, D), jnp.float32),
        jax.ShapeDtypeStruct((NREL, NPAD // 4, 4 * HALF), jnp.float32),
        jax.ShapeDtypeStruct((NREL, NPAD // 4, 4 * HALF), jnp.float32),
    ]
    return pl.pallas_call(_blk_mid_body, grid=(GRID,), in_specs=in_specs,
                          out_specs=out_specs, out_shape=out_shape)


CPK = 56
NKC = 14


def _make_sc_agg(epad):
    nch = epad // (SUB * CH)
    assert nch == CPK * NKC
    rpt = NPAD // SUB
    mesh = plsc.VectorSubcoreMesh(core_axis_name="c", subcore_axis_name="s")

    @functools.partial(
        pl.kernel, mesh=mesh,
        compiler_params=pltpu.CompilerParams(use_tc_tiling_on_sc=False),
        out_type=(jax.ShapeDtypeStruct((NPAD, HALF), jnp.float32),
                  jax.ShapeDtypeStruct((NPAD, HALF), jnp.float32)),
        scratch_types=[
            pltpu.VMEM((CPK, CH), jnp.int32),
            pltpu.VMEM((CPK, CH), jnp.int32),
            pltpu.VMEM((2, CH, HALF), jnp.float32),
            pltpu.VMEM_SHARED((NPAD, HALF), jnp.float32),
            pltpu.SemaphoreType.DMA,
            pltpu.SemaphoreType.DMA,
        ],
    )
    def sc_agg(u_hbm, v_hbm, ya_hbm, yb_hbm, zero_hbm, outa_hbm, outb_hbm,
               u_k, v_k, rows_v, acc, sem0, sem1):
        c = lax.axis_index("c")
        s = lax.axis_index("s")
        rbase = s * rpt
        crow_base = s * nch

        def run(y_t, out_t):
            pltpu.sync_copy(zero_hbm.at[pl.ds(rbase, rpt)],
                            acc.at[pl.ds(rbase, rpt)])
            plsc.subcore_barrier()

            sems = (sem0, sem1)
            for kc in range(NKC):
                pltpu.sync_copy(u_hbm.at[pl.ds(crow_base + kc * CPK, CPK)], u_k)
                pltpu.sync_copy(v_hbm.at[pl.ds(crow_base + kc * CPK, CPK)], v_k)
                pltpu.async_copy(y_t.at[v_k.at[0]], rows_v.at[0], sem0)

                def inner(t, carry):
                    for b in (0, 1):
                        j = t * 2 + b
                        pltpu.make_async_copy(y_t.at[pl.ds(0, CH)],
                                              rows_v.at[b], sems[b]).wait()

                        @pl.when(j + 1 < CPK)
                        def _():
                            pltpu.async_copy(y_t.at[v_k.at[j + 1]],
                                             rows_v.at[1 - b], sems[1 - b])

                        pltpu.sync_copy(rows_v.at[b], acc.at[u_k.at[j]],
                                        add=True)
                    return carry

                lax.fori_loop(0, CPK // 2, inner, 0)

            plsc.subcore_barrier()
            pltpu.sync_copy(acc.at[pl.ds(rbase, rpt)],
                            out_t.at[pl.ds(rbase, rpt)])

        @pl.when(c == 0)
        def _():
            run(ya_hbm, outa_hbm)

        @pl.when(c == 1)
        def _():
            run(yb_hbm, outb_hbm)

    return sc_agg


def kernel(feats, ctrs, W_in1, b_in1, W_in2, g_in, be_in,
           W_seg1, b_seg1, W_seg2, g_seg, be_seg,
           W_ctr, W_pre0, W_pre1, W_suc0, W_suc1, W_left, W_right, W_ctr2,
           g_norm, be_norm, g_ctr2, be_ctr2,
           pre0_u, pre0_v, suc0_u, suc0_v, pre1_u, pre1_v, suc1_u, suc1_v,
           left_u, left_v, right_u, right_v):
    f32 = jnp.float32
    r2 = lambda a: a.reshape(1, D).astype(f32)

    us = [pre0_u, suc0_u, pre1_u, suc1_u, left_u, right_u]
    vs = [pre0_v, suc0_v, pre1_v, suc1_v, left_v, right_v]
    e_tot = sum(int(u.shape[0]) for u in us)
    align = SUB * CH * CPK
    epad = ((e_tot + align - 1) // align) * align
    pad = epad - e_tot
    u_all = jnp.concatenate(
        [u.astype(jnp.int32) for u in us]
        + [jnp.full((pad,), N, jnp.int32)]).reshape(epad // CH, CH)
    def enc(v, r):
        v = v.astype(jnp.int32)
        l = v % jnp.int32(RBLK)
        return (jnp.int32(r * NPAD) + (v - l) + 4 * (l % jnp.int32(RBLK // 4))
                + l // jnp.int32(RBLK // 4))

    v_all = jnp.concatenate(
        [enc(v, r) for r, v in enumerate(vs)]
        + [jnp.zeros((pad,), jnp.int32)]).reshape(epad // CH, CH)
    zero = jnp.zeros((NPAD, HALF), jnp.float32)

    ctrs_p = jnp.zeros((NPAD, 2), f32).at[:N].set(ctrs)
    feats_p = jnp.zeros((NPAD, 2), f32).at[:N].set(feats)

    def wall(i):
        return jnp.concatenate(
            [W_ctr[i], W_pre0[i], W_suc0[i], W_pre1[i], W_suc1[i],
             W_left[i], W_right[i]], axis=1)

    k1 = _make_k1()
    blk_mid = _make_blk(last=False)
    blk_last = _make_blk(last=True)
    sc_agg = _make_sc_agg(epad)

    feat0, t00, ya, yb = k1(
        ctrs_p, feats_p, W_in1, r2(b_in1), W_in2, r2(g_in), r2(be_in),
        W_seg1, r2(b_seg1), W_seg2, r2(g_seg), r2(be_seg), wall(0))

    s0a, s0b = sc_agg(u_all, v_all,
                      ya.reshape(NREL * NPAD, HALF),
                      yb.reshape(NREL * NPAD, HALF), zero)

    feat1, t01, y1a, y1b = blk_mid(
        s0a, s0b, t00, feat0, r2(g_norm[0]), r2(be_norm[0]), W_ctr2[0],
        r2(g_ctr2[0]), r2(be_ctr2[0]), wall(1))

    s1a, s1b = sc_agg(u_all, v_all,
                      y1a.reshape(NREL * NPAD, HALF),
                      y1b.reshape(NREL * NPAD, HALF), zero)

    out = blk_last(s1a, s1b, t01, feat1, r2(g_norm[1]), r2(be_norm[1]),
                   W_ctr2[1], r2(g_ctr2[1]), r2(be_ctr2[1]))
    return out
